# Initial kernel scaffold; baseline (speedup 1.0000x reference)
import jax
import jax.numpy as jnp
from jax.experimental import pallas as pl


def kernel(x, add_table, mult_table):
    raise NotImplementedError("write your pallas kernel here")


# speedup vs baseline: 1.0661x; 1.0661x over previous
import functools

import jax
import jax.numpy as jnp
from jax.experimental import pallas as pl
from jax.experimental.pallas import tpu as pltpu
from jax.experimental.pallas import tpu_sc as plsc

_LANES = 16
_BR = 4


def kernel(x, add_table, mult_table):
    B, L, D = x.shape
    mesh = plsc.VectorSubcoreMesh(core_axis_name="core",
                                  subcore_axis_name="subcore")

    @functools.partial(
        pl.kernel,
        out_type=jax.ShapeDtypeStruct((B, L, D), x.dtype),
        mesh=mesh,
    )
    def run(x_hbm, add_hbm, mult_hbm, o_hbm):
        def body(x_v, add_v, mult_v, o_v):
            @pl.loop(0, _BR)
            def _row(r):
                @pl.loop(0, D, step=_LANES)
                def _col(c):
                    sl = pl.ds(c, _LANES)
                    a = add_v.at[r, sl][...]
                    m = mult_v.at[r, sl][...]
                    for b in range(B):
                        o_v.at[b, r, sl][...] = x_v.at[b, r, sl][...] * m + a

        pltpu.emit_pipeline(
            body,
            grid=(L // _BR,),
            in_specs=[
                pl.BlockSpec((B, _BR, D), lambda i: (0, i, 0)),
                pl.BlockSpec((_BR, D), lambda i: (i, 0)),
                pl.BlockSpec((_BR, D), lambda i: (i, 0)),
            ],
            out_specs=[pl.BlockSpec((B, _BR, D), lambda i: (0, i, 0))],
            core_axis_name=("core", "subcore"),
            dimension_semantics=(pltpu.PARALLEL,),
        )(x_hbm, add_hbm, mult_hbm, o_hbm)

    return run(x, add_table, mult_table)
